# baseline (speedup 1.0000x reference)
import functools
import jax
import jax.numpy as jnp
from jax import lax
from jax.experimental import pallas as pl
from jax.experimental.pallas import tpu as pltpu
from jax.experimental.pallas import tpu_sc as plsc


_GATHER_G = 8
_GATHER_LANES = 128


def _gcd(a, b):
    while b:
        a, b = b, a % b
    return a


def _sc_gather(table, idx_pad2, groups_per_worker):
    v, d = table.shape
    r = idx_pad2.shape[0]
    nw = 32
    rows_buf = _GATHER_G * _GATHER_LANES

    mesh = plsc.VectorSubcoreMesh(core_axis_name="c", subcore_axis_name="s")

    @functools.partial(
        pl.kernel,
        out_type=jax.ShapeDtypeStruct((r * _GATHER_LANES, d), jnp.float32),
        mesh=mesh,
        compiler_params=pltpu.CompilerParams(use_tc_tiling_on_sc=False),
        scratch_types=[
            pltpu.VMEM((_GATHER_G, _GATHER_LANES), jnp.int32),
            pltpu.VMEM((rows_buf, d), jnp.float32),
            pltpu.SemaphoreType.DMA,
        ],
    )
    def k(table_hbm, idx_hbm, out_hbm, idx_v, rows_v, sem):
        nc = 2
        wid = lax.axis_index("s") * nc + lax.axis_index("c")
        base = wid * (groups_per_worker * _GATHER_G)

        def body(g, carry):
            row0 = base + g * _GATHER_G
            pltpu.sync_copy(idx_hbm.at[pl.ds(row0, _GATHER_G)], idx_v)
            copies = []
            for j in range(_GATHER_G):
                copies.append(pltpu.async_copy(
                    table_hbm.at[idx_v.at[j]],
                    rows_v.at[pl.ds(j * _GATHER_LANES, _GATHER_LANES)],
                    sem))
            for c in copies:
                c.wait()
            pltpu.sync_copy(rows_v,
                            out_hbm.at[pl.ds(row0 * _GATHER_LANES, rows_buf)])
            return carry

        lax.fori_loop(0, groups_per_worker, body, None)

    return k(table, idx_pad2)


def _enc_body(x_ref, w_ref, o_ref):
    o_ref[...] = jnp.maximum(x_ref[...] @ w_ref[...], 0.0)


def _encode(features, W_pw, bm):
    n, k = features.shape
    d = W_pw.shape[1]
    return pl.pallas_call(
        _enc_body,
        grid=(n // bm,),
        in_specs=[
            pl.BlockSpec((bm, k), lambda i: (i, 0)),
            pl.BlockSpec((k, d), lambda i: (0, 0)),
        ],
        out_specs=pl.BlockSpec((bm, d), lambda i: (i, 0)),
        out_shape=jax.ShapeDtypeStruct((n, d), jnp.float32),
    )(features, W_pw)


def _prop_body(a_ref, x_ref, o_ref):
    o_ref[...] = a_ref[...] @ x_ref[...]


def _prop2_body(a_ref, x_ref, e_ref, t_ref, o_ref):
    o_ref[...] = (a_ref[...] @ x_ref[...] + e_ref[...] + t_ref[...]) * (1.0 / 3.0)


def _propagate(adj, emb0, bm):
    n, d = emb0.shape
    t1 = pl.pallas_call(
        _prop_body,
        grid=(n // bm,),
        in_specs=[
            pl.BlockSpec((bm, n), lambda i: (i, 0)),
            pl.BlockSpec((n, d), lambda i: (0, 0)),
        ],
        out_specs=pl.BlockSpec((bm, d), lambda i: (i, 0)),
        out_shape=jax.ShapeDtypeStruct((n, d), jnp.float32),
    )(adj, emb0)
    gnn = pl.pallas_call(
        _prop2_body,
        grid=(n // bm,),
        in_specs=[
            pl.BlockSpec((bm, n), lambda i: (i, 0)),
            pl.BlockSpec((n, d), lambda i: (0, 0)),
            pl.BlockSpec((bm, d), lambda i: (i, 0)),
            pl.BlockSpec((bm, d), lambda i: (i, 0)),
        ],
        out_specs=pl.BlockSpec((bm, d), lambda i: (i, 0)),
        out_shape=jax.ShapeDtypeStruct((n, d), jnp.float32),
    )(adj, t1, emb0, t1)
    return gnn


def _lstm_body(pe_ref, len_ref, wih_ref, whh_ref, b_ref, wo_ref, bo_ref,
               o_ref, *, nl, h_dim):
    x = pe_ref[...]
    bp = x.shape[0]
    b = b_ref[...]
    wcat = jnp.concatenate([wih_ref[...], whh_ref[...]], axis=0)
    idx = jnp.clip(len_ref[...] - 1, 0, nl - 1)
    h = jnp.zeros((bp, h_dim), jnp.float32)
    c = jnp.zeros((bp, h_dim), jnp.float32)
    h_last = jnp.zeros((bp, h_dim), jnp.float32)
    d = x.shape[1] // nl
    wcat_b = wcat.astype(jnp.bfloat16)
    for l in range(nl):
        x_t = x[:, l * d:(l + 1) * d]
        xh = jnp.concatenate([x_t, h], axis=1).astype(jnp.bfloat16)
        z = jnp.dot(xh, wcat_b, preferred_element_type=jnp.float32) + b
        i_g = jax.nn.sigmoid(z[:, :h_dim])
        f_g = jax.nn.sigmoid(z[:, h_dim:2 * h_dim])
        g_g = jnp.tanh(z[:, 2 * h_dim:3 * h_dim])
        o_g = jax.nn.sigmoid(z[:, 3 * h_dim:])
        c = f_g * c + i_g * g_g
        h = o_g * jnp.tanh(c)
        h_last = jnp.where(idx == l, h, h_last)
    pw = jax.nn.sigmoid(h_last @ wo_ref[...] + bo_ref[0, 0])
    o_ref[...] = pw


def _lstm_pw(path_emb, lengths, W_ih, W_hh, b_ih, b_hh, w_out, b_out, bp):
    p, ld = path_emb.shape
    h_dim = W_hh.shape[1]
    nl = ld // (W_ih.shape[1])
    wih = W_ih.T
    whh = W_hh.T
    b = (b_ih + b_hh).reshape(1, -1)
    wo = w_out.reshape(-1, 1)
    bo = b_out.reshape(1, 1)
    lengths2 = lengths.reshape(p, 1)
    pw2 = pl.pallas_call(
        functools.partial(_lstm_body, nl=nl, h_dim=h_dim),
        grid=(p // bp,),
        in_specs=[
            pl.BlockSpec((bp, ld), lambda i: (i, 0)),
            pl.BlockSpec((bp, 1), lambda i: (i, 0)),
            pl.BlockSpec(wih.shape, lambda i: (0, 0)),
            pl.BlockSpec(whh.shape, lambda i: (0, 0)),
            pl.BlockSpec(b.shape, lambda i: (0, 0)),
            pl.BlockSpec(wo.shape, lambda i: (0, 0)),
            pl.BlockSpec(bo.shape, lambda i: (0, 0)),
        ],
        out_specs=pl.BlockSpec((bp, 1), lambda i: (i, 0)),
        out_shape=jax.ShapeDtypeStruct((p, 1), jnp.float32),
    )(path_emb, lengths2, wih, whh, b, wo, bo)
    return pw2.reshape(p)


def _smax_body(a_ref, g_ref, o_ref, e_ref):
    a = a_ref[...]
    aw = jnp.where(a > 0.0, a, jnp.float32(-9e15))
    m = jnp.max(aw, axis=1, keepdims=True)
    ex = jnp.exp(aw - m)
    s = jnp.sum(ex, axis=1, keepdims=True)
    p = ex / s
    o_ref[...] = p
    e_ref[...] = p @ g_ref[...]


def _softmax_spmm(A, gnn, bm):
    n = A.shape[0]
    d = gnn.shape[1]
    return pl.pallas_call(
        _smax_body,
        grid=(n // bm,),
        in_specs=[
            pl.BlockSpec((bm, n), lambda i: (i, 0)),
            pl.BlockSpec((n, d), lambda i: (0, 0)),
        ],
        out_specs=[
            pl.BlockSpec((bm, n), lambda i: (i, 0)),
            pl.BlockSpec((bm, d), lambda i: (i, 0)),
        ],
        out_shape=[
            jax.ShapeDtypeStruct((n, n), jnp.float32),
            jax.ShapeDtypeStruct((n, d), jnp.float32),
        ],
    )(A, gnn)


def _head_body(g_ref, pe_ref, w1_ref, b1_ref, w2_ref, b2_ref, o_ref, *, lam):
    e = jnp.concatenate([g_ref[...], lam * pe_ref[...]], axis=1)
    h = jnp.maximum(e @ w1_ref[...] + b1_ref[...], 0.0)
    lg = h @ w2_ref[...] + b2_ref[...]
    m = jnp.max(lg, axis=1, keepdims=True)
    lse = m + jnp.log(jnp.sum(jnp.exp(lg - m), axis=1, keepdims=True))
    o_ref[...] = lg - lse


def _head(gnn, pw_emd, W1, b1, W2, b2, lam, bm):
    n, d = gnn.shape
    nh = W1.shape[1]
    nc = W2.shape[1]
    return pl.pallas_call(
        functools.partial(_head_body, lam=lam),
        grid=(n // bm,),
        in_specs=[
            pl.BlockSpec((bm, d), lambda i: (i, 0)),
            pl.BlockSpec((bm, d), lambda i: (i, 0)),
            pl.BlockSpec(W1.shape, lambda i: (0, 0)),
            pl.BlockSpec((1, nh), lambda i: (0, 0)),
            pl.BlockSpec(W2.shape, lambda i: (0, 0)),
            pl.BlockSpec((1, nc), lambda i: (0, 0)),
        ],
        out_specs=pl.BlockSpec((bm, nc), lambda i: (i, 0)),
        out_shape=jax.ShapeDtypeStruct((n, nc), jnp.float32),
    )(gnn, pw_emd, W1, b1.reshape(1, -1), W2, b2.reshape(1, -1))


def kernel(features, adj, pairs, sub_paths, sub_path_length, W_pw, W_ih, W_hh,
           b_ih, b_hh, w_out, b_out, W1, b1, W2, b2):
    n = features.shape[0]
    d = W_pw.shape[1]
    p, l = sub_paths.shape

    bm_enc = 2000 if n % 2000 == 0 else n
    emb0 = _encode(features, W_pw, bm_enc)

    bm = 1000 if n % 1000 == 0 else n
    bmp = 200 if n % 200 == 0 else n
    gnn = _propagate(adj, emb0, bmp)

    flat_idx = sub_paths.reshape(-1).astype(jnp.int32)
    unit = 32 * _GATHER_G * _GATHER_LANES
    flat_unit = (unit // _gcd(unit, l)) * l
    npad = ((p * l + flat_unit - 1) // flat_unit) * flat_unit
    idx_pad = jnp.pad(flat_idx, (0, npad - p * l))
    p_pad = npad // l
    lengths = jnp.pad(sub_path_length.astype(jnp.int32), (0, p_pad - p))
    nunits = npad // flat_unit
    nchunks = 1
    for cand in (5, 2):
        if nunits % cand == 0:
            nchunks = cand
            break
    cflat = npad // nchunks
    cpaths = p_pad // nchunks
    bp = 2048 if cpaths % 2048 == 0 else (2000 if cpaths % 2000 == 0 else cpaths)
    pw_parts = []
    for ci in range(nchunks):
        idx2 = idx_pad[ci * cflat:(ci + 1) * cflat].reshape(-1, _GATHER_LANES)
        rows = _sc_gather(gnn, idx2, cflat // unit)
        pe = rows.reshape(cpaths, l * d)
        ln = lengths[ci * cpaths:(ci + 1) * cpaths]
        pw_parts.append(_lstm_pw(pe, ln, W_ih, W_hh, b_ih, b_hh,
                                 w_out, b_out, bp))
    pw = jnp.concatenate(pw_parts)[:p]

    A = jnp.zeros((n, n), jnp.float32).at[pairs[:, 0], pairs[:, 1]].add(pw)
    diag = jnp.arange(n)
    A = A.at[diag, diag].add(1.0)

    bs = 200 if n % 200 == 0 else n
    pw_adj, pw_emd = _softmax_spmm(A, gnn, bs)

    logp = _head(gnn, pw_emd, W1, b1, W2, b2, 1.0, bm)
    return (logp, pw_adj)


# speedup vs baseline: 1.0028x; 1.0028x over previous
import functools
import jax
import jax.numpy as jnp
from jax import lax
from jax.experimental import pallas as pl
from jax.experimental.pallas import tpu as pltpu
from jax.experimental.pallas import tpu_sc as plsc


_GATHER_G = 8
_GATHER_LANES = 128


def _gcd(a, b):
    while b:
        a, b = b, a % b
    return a


def _sc_gather(table, idx_pad2, groups_per_worker):
    v, d = table.shape
    r = idx_pad2.shape[0]
    nw = 32
    rows_buf = _GATHER_G * _GATHER_LANES

    mesh = plsc.VectorSubcoreMesh(core_axis_name="c", subcore_axis_name="s")

    @functools.partial(
        pl.kernel,
        out_type=jax.ShapeDtypeStruct((r * _GATHER_LANES, d), jnp.float32),
        mesh=mesh,
        compiler_params=pltpu.CompilerParams(use_tc_tiling_on_sc=False),
        scratch_types=[
            pltpu.VMEM((_GATHER_G, _GATHER_LANES), jnp.int32),
            pltpu.VMEM((rows_buf, d), jnp.float32),
            pltpu.SemaphoreType.DMA,
        ],
    )
    def k(table_hbm, idx_hbm, out_hbm, idx_v, rows_v, sem):
        nc = 2
        wid = lax.axis_index("s") * nc + lax.axis_index("c")
        base = wid * (groups_per_worker * _GATHER_G)

        def body(g, carry):
            row0 = base + g * _GATHER_G
            pltpu.sync_copy(idx_hbm.at[pl.ds(row0, _GATHER_G)], idx_v)
            copies = []
            for j in range(_GATHER_G):
                copies.append(pltpu.async_copy(
                    table_hbm.at[idx_v.at[j]],
                    rows_v.at[pl.ds(j * _GATHER_LANES, _GATHER_LANES)],
                    sem))
            for c in copies:
                c.wait()
            pltpu.sync_copy(rows_v,
                            out_hbm.at[pl.ds(row0 * _GATHER_LANES, rows_buf)])
            return carry

        lax.fori_loop(0, groups_per_worker, body, None)

    return k(table, idx_pad2)


def _enc_body(x_ref, w_ref, o_ref):
    o_ref[...] = jnp.maximum(x_ref[...] @ w_ref[...], 0.0)


def _encode(features, W_pw, bm):
    n, k = features.shape
    d = W_pw.shape[1]
    return pl.pallas_call(
        _enc_body,
        grid=(n // bm,),
        in_specs=[
            pl.BlockSpec((bm, k), lambda i: (i, 0)),
            pl.BlockSpec((k, d), lambda i: (0, 0)),
        ],
        out_specs=pl.BlockSpec((bm, d), lambda i: (i, 0)),
        out_shape=jax.ShapeDtypeStruct((n, d), jnp.float32),
    )(features, W_pw)


def _prop_body(a_ref, x_ref, o_ref):
    o_ref[...] = a_ref[...] @ x_ref[...]


def _prop2_body(a_ref, x_ref, e_ref, t_ref, o_ref):
    o_ref[...] = (a_ref[...] @ x_ref[...] + e_ref[...] + t_ref[...]) * (1.0 / 3.0)


def _propagate(adj, emb0, bm):
    n, d = emb0.shape
    t1 = pl.pallas_call(
        _prop_body,
        grid=(n // bm,),
        in_specs=[
            pl.BlockSpec((bm, n), lambda i: (i, 0)),
            pl.BlockSpec((n, d), lambda i: (0, 0)),
        ],
        out_specs=pl.BlockSpec((bm, d), lambda i: (i, 0)),
        out_shape=jax.ShapeDtypeStruct((n, d), jnp.float32),
    )(adj, emb0)
    gnn = pl.pallas_call(
        _prop2_body,
        grid=(n // bm,),
        in_specs=[
            pl.BlockSpec((bm, n), lambda i: (i, 0)),
            pl.BlockSpec((n, d), lambda i: (0, 0)),
            pl.BlockSpec((bm, d), lambda i: (i, 0)),
            pl.BlockSpec((bm, d), lambda i: (i, 0)),
        ],
        out_specs=pl.BlockSpec((bm, d), lambda i: (i, 0)),
        out_shape=jax.ShapeDtypeStruct((n, d), jnp.float32),
    )(adj, t1, emb0, t1)
    return gnn


def _lstm_body(pe_ref, len_ref, wih_ref, whh_ref, b_ref, wo_ref, bo_ref,
               o_ref, *, nl, h_dim):
    x = pe_ref[...]
    bp = x.shape[0]
    b = b_ref[...]
    wcat = jnp.concatenate([wih_ref[...], whh_ref[...]], axis=0)
    idx = jnp.clip(len_ref[...] - 1, 0, nl - 1)
    h = jnp.zeros((bp, h_dim), jnp.float32)
    c = jnp.zeros((bp, h_dim), jnp.float32)
    h_last = jnp.zeros((bp, h_dim), jnp.float32)
    d = x.shape[1] // nl
    for l in range(nl):
        x_t = x[:, l * d:(l + 1) * d]
        z = jnp.concatenate([x_t, h], axis=1) @ wcat + b
        i_g = jax.nn.sigmoid(z[:, :h_dim])
        f_g = jax.nn.sigmoid(z[:, h_dim:2 * h_dim])
        g_g = jnp.tanh(z[:, 2 * h_dim:3 * h_dim])
        o_g = jax.nn.sigmoid(z[:, 3 * h_dim:])
        c = f_g * c + i_g * g_g
        h = o_g * jnp.tanh(c)
        h_last = jnp.where(idx == l, h, h_last)
    pw = jax.nn.sigmoid(h_last @ wo_ref[...] + bo_ref[0, 0])
    o_ref[...] = pw


def _lstm_pw(path_emb, lengths, W_ih, W_hh, b_ih, b_hh, w_out, b_out, bp):
    p, ld = path_emb.shape
    h_dim = W_hh.shape[1]
    nl = ld // (W_ih.shape[1])
    wih = W_ih.T
    whh = W_hh.T
    b = (b_ih + b_hh).reshape(1, -1)
    wo = w_out.reshape(-1, 1)
    bo = b_out.reshape(1, 1)
    lengths2 = lengths.reshape(p, 1)
    pw2 = pl.pallas_call(
        functools.partial(_lstm_body, nl=nl, h_dim=h_dim),
        grid=(p // bp,),
        in_specs=[
            pl.BlockSpec((bp, ld), lambda i: (i, 0)),
            pl.BlockSpec((bp, 1), lambda i: (i, 0)),
            pl.BlockSpec(wih.shape, lambda i: (0, 0)),
            pl.BlockSpec(whh.shape, lambda i: (0, 0)),
            pl.BlockSpec(b.shape, lambda i: (0, 0)),
            pl.BlockSpec(wo.shape, lambda i: (0, 0)),
            pl.BlockSpec(bo.shape, lambda i: (0, 0)),
        ],
        out_specs=pl.BlockSpec((bp, 1), lambda i: (i, 0)),
        out_shape=jax.ShapeDtypeStruct((p, 1), jnp.float32),
    )(path_emb, lengths2, wih, whh, b, wo, bo)
    return pw2.reshape(p)


def _smax_body(a_ref, g_ref, o_ref, e_ref):
    a = a_ref[...]
    aw = jnp.where(a > 0.0, a, jnp.float32(-9e15))
    m = jnp.max(aw, axis=1, keepdims=True)
    ex = jnp.exp(aw - m)
    s = jnp.sum(ex, axis=1, keepdims=True)
    p = ex / s
    o_ref[...] = p
    e_ref[...] = p @ g_ref[...]


def _softmax_spmm(A, gnn, bm):
    n = A.shape[0]
    d = gnn.shape[1]
    return pl.pallas_call(
        _smax_body,
        grid=(n // bm,),
        in_specs=[
            pl.BlockSpec((bm, n), lambda i: (i, 0)),
            pl.BlockSpec((n, d), lambda i: (0, 0)),
        ],
        out_specs=[
            pl.BlockSpec((bm, n), lambda i: (i, 0)),
            pl.BlockSpec((bm, d), lambda i: (i, 0)),
        ],
        out_shape=[
            jax.ShapeDtypeStruct((n, n), jnp.float32),
            jax.ShapeDtypeStruct((n, d), jnp.float32),
        ],
    )(A, gnn)


def _head_body(g_ref, pe_ref, w1_ref, b1_ref, w2_ref, b2_ref, o_ref, *, lam):
    e = jnp.concatenate([g_ref[...], lam * pe_ref[...]], axis=1)
    h = jnp.maximum(e @ w1_ref[...] + b1_ref[...], 0.0)
    lg = h @ w2_ref[...] + b2_ref[...]
    m = jnp.max(lg, axis=1, keepdims=True)
    lse = m + jnp.log(jnp.sum(jnp.exp(lg - m), axis=1, keepdims=True))
    o_ref[...] = lg - lse


def _head(gnn, pw_emd, W1, b1, W2, b2, lam, bm):
    n, d = gnn.shape
    nh = W1.shape[1]
    nc = W2.shape[1]
    return pl.pallas_call(
        functools.partial(_head_body, lam=lam),
        grid=(n // bm,),
        in_specs=[
            pl.BlockSpec((bm, d), lambda i: (i, 0)),
            pl.BlockSpec((bm, d), lambda i: (i, 0)),
            pl.BlockSpec(W1.shape, lambda i: (0, 0)),
            pl.BlockSpec((1, nh), lambda i: (0, 0)),
            pl.BlockSpec(W2.shape, lambda i: (0, 0)),
            pl.BlockSpec((1, nc), lambda i: (0, 0)),
        ],
        out_specs=pl.BlockSpec((bm, nc), lambda i: (i, 0)),
        out_shape=jax.ShapeDtypeStruct((n, nc), jnp.float32),
    )(gnn, pw_emd, W1, b1.reshape(1, -1), W2, b2.reshape(1, -1))


def kernel(features, adj, pairs, sub_paths, sub_path_length, W_pw, W_ih, W_hh,
           b_ih, b_hh, w_out, b_out, W1, b1, W2, b2):
    n = features.shape[0]
    d = W_pw.shape[1]
    p, l = sub_paths.shape

    bm_enc = 2000 if n % 2000 == 0 else n
    emb0 = _encode(features, W_pw, bm_enc)

    bm = 1000 if n % 1000 == 0 else n
    bmp = 200 if n % 200 == 0 else n
    gnn = _propagate(adj, emb0, bmp)

    flat_idx = sub_paths.reshape(-1).astype(jnp.int32)
    unit = 32 * _GATHER_G * _GATHER_LANES
    flat_unit = (unit // _gcd(unit, l)) * l
    npad = ((p * l + flat_unit - 1) // flat_unit) * flat_unit
    idx_pad = jnp.pad(flat_idx, (0, npad - p * l))
    p_pad = npad // l
    lengths = jnp.pad(sub_path_length.astype(jnp.int32), (0, p_pad - p))
    nunits = npad // flat_unit
    nchunks = 1
    for cand in (5, 2):
        if nunits % cand == 0:
            nchunks = cand
            break
    cflat = npad // nchunks
    cpaths = p_pad // nchunks
    bp = 2048 if cpaths % 2048 == 0 else (2000 if cpaths % 2000 == 0 else cpaths)
    pw_parts = []
    for ci in range(nchunks):
        idx2 = idx_pad[ci * cflat:(ci + 1) * cflat].reshape(-1, _GATHER_LANES)
        rows = _sc_gather(gnn, idx2, cflat // unit)
        pe = rows.reshape(cpaths, l * d)
        ln = lengths[ci * cpaths:(ci + 1) * cpaths]
        pw_parts.append(_lstm_pw(pe, ln, W_ih, W_hh, b_ih, b_hh,
                                 w_out, b_out, bp))
    pw = jnp.concatenate(pw_parts)[:p]

    A = jnp.zeros((n, n), jnp.float32).at[pairs[:, 0], pairs[:, 1]].add(pw)
    diag = jnp.arange(n)
    A = A.at[diag, diag].add(1.0)

    bs = 200 if n % 200 == 0 else n
    pw_adj, pw_emd = _softmax_spmm(A, gnn, bs)

    logp = _head(gnn, pw_emd, W1, b1, W2, b2, 1.0, bm)
    return (logp, pw_adj)
